# baseline (speedup 1.0000x reference)
import functools

import jax
import jax.numpy as jnp
from jax import lax
from jax.experimental import pallas as pl
from jax.experimental.pallas import tpu as pltpu
from jax.experimental.pallas import tpu_sc as plsc

NC = 2
NS = 16
L = 16
NW = NC * NS

D = 128
ND = D // L
IDS_PAD = 192
HALF = IDS_PAD // 2
BLK = 64


def _sc_embed(table, ids3, warr, B):
    rpw = B // NW
    nblk = rpw // BLK
    mesh = plsc.VectorSubcoreMesh(core_axis_name="c", subcore_axis_name="s")

    @functools.partial(
        pl.kernel,
        out_type=jax.ShapeDtypeStruct((B, D), jnp.float32),
        mesh=mesh,
        scratch_types=[
            pltpu.VMEM((BLK, 2, HALF), jnp.int32),
            pltpu.VMEM((IDS_PAD, D), jnp.float32),
            pltpu.VMEM((IDS_PAD, D), jnp.float32),
            pltpu.VMEM((4, L), jnp.float32),
            pltpu.VMEM((BLK, D), jnp.float32),
            pltpu.SemaphoreType.DMA,
            pltpu.SemaphoreType.DMA,
        ],
    )
    def k(table_hbm, ids_hbm, warr_hbm, out_hbm, ids_v, rows_a, rows_b, w_v,
          out_v, sem_a, sem_b):
        wid = lax.axis_index("s") * NC + lax.axis_index("c")
        base = wid * rpw

        pltpu.sync_copy(warr_hbm, w_v)
        w0 = w_v[0, :]
        w1 = w_v[1, :]
        w2 = w_v[2, :]
        w3 = w_v[3, :]

        zero = jnp.zeros((L,), jnp.float32)

        def start_gather(rr, buf, sem):
            pltpu.async_copy(
                table_hbm.at[ids_v.at[rr, 0]], buf.at[pl.ds(0, HALF)], sem)
            pltpu.async_copy(
                table_hbm.at[ids_v.at[rr, 1]], buf.at[pl.ds(HALF, HALF)], sem)

        def wait_gather(buf, sem):
            pltpu.make_async_copy(
                table_hbm.at[pl.ds(0, IDS_PAD)], buf, sem).wait()

        def field_sum(buf, lo, hi, unroll):
            n = (hi - lo) // unroll

            def jbody(t, accs):
                j = lo + t * unroll
                accs = list(accs)
                for u in range(unroll):
                    for d in range(ND):
                        accs[d] = accs[d] + buf[j + u, pl.ds(d * L, L)]
                return tuple(accs)

            return lax.fori_loop(0, n, jbody, (zero,) * ND)

        def accum_row(buf, rr):
            s0 = field_sum(buf, 0, 20, 5)
            s1 = field_sum(buf, 20, 40, 5)
            s2 = field_sum(buf, 40, 90, 5)
            s3 = field_sum(buf, 90, IDS_PAD, 6)
            for d in range(ND):
                out_v[rr, pl.ds(d * L, L)] = (
                    s0[d] * w0 + s1[d] * w1 + s2[d] * w2 + s3[d] * w3)

        def pair_body(kk, carry):
            start_gather(2 * kk + 1, rows_b, sem_b)
            wait_gather(rows_a, sem_a)
            accum_row(rows_a, 2 * kk)

            @pl.when(kk < BLK // 2 - 1)
            def _():
                start_gather(2 * kk + 2, rows_a, sem_a)

            wait_gather(rows_b, sem_b)
            accum_row(rows_b, 2 * kk + 1)
            return carry

        def blk_body(blk, carry):
            pltpu.sync_copy(ids_hbm.at[pl.ds(base + blk * BLK, BLK)], ids_v)
            start_gather(0, rows_a, sem_a)
            lax.fori_loop(0, BLK // 2, pair_body, 0)
            pltpu.sync_copy(out_v, out_hbm.at[pl.ds(base + blk * BLK, BLK)])
            return carry

        lax.fori_loop(0, nblk, blk_body, 0)

    return k(table, ids3, warr)


def _tc_normalize(e):
    B = e.shape[0]
    blk = 2048

    def nrm(e_ref, o_ref):
        x = e_ref[...]
        n = jnp.sqrt(jnp.sum(x * x, axis=1, keepdims=True))
        o_ref[...] = x / jnp.maximum(n, 1e-12)

    return pl.pallas_call(
        nrm,
        out_shape=jax.ShapeDtypeStruct((B, D), jnp.float32),
        grid=(B // blk,),
        in_specs=[pl.BlockSpec((blk, D), lambda i: (i, 0))],
        out_specs=pl.BlockSpec((blk, D), lambda i: (i, 0)),
    )(e)


def kernel(tags_ids, component_ids, summary_ids, body_ids, token_embedding,
           field_weights):
    B = tags_ids.shape[0]
    lens = (tags_ids.shape[1], component_ids.shape[1], summary_ids.shape[1],
            body_ids.shape[1])
    pad = IDS_PAD - sum(lens)
    ids = jnp.concatenate(
        [tags_ids.astype(jnp.int32), component_ids.astype(jnp.int32),
         summary_ids.astype(jnp.int32), body_ids.astype(jnp.int32),
         jnp.zeros((B, pad), jnp.int32)], axis=1)
    ids3 = ids.reshape(B, 2, HALF)
    scales = field_weights.astype(jnp.float32) / jnp.array(lens, jnp.float32)
    warr = jnp.broadcast_to(scales[:, None], (4, L))
    e = _sc_embed(token_embedding, ids3, warr, B)
    return _tc_normalize(e)


# speedup vs baseline: 2.4555x; 2.4555x over previous
import functools

import jax
import jax.numpy as jnp
from jax import lax
from jax.experimental import pallas as pl
from jax.experimental.pallas import tpu as pltpu
from jax.experimental.pallas import tpu_sc as plsc

NC = 2
NS = 16
L = 16
NW = NC * NS

D = 128
ND = D // L
NIDS = 190
IDS_PAD = 192
HALF = IDS_PAD // 2
BLK = 64


def _sc_embed(table, ids3, warr, B):
    rpw = B // NW
    nblk = rpw // BLK
    mesh = plsc.VectorSubcoreMesh(core_axis_name="c", subcore_axis_name="s")

    @functools.partial(
        pl.kernel,
        out_type=jax.ShapeDtypeStruct((B, D), jnp.float32),
        mesh=mesh,
        scratch_types=[
            pltpu.VMEM((BLK, 2, HALF), jnp.int32),
            pltpu.VMEM((IDS_PAD, D), jnp.float32),
            pltpu.VMEM((IDS_PAD, D), jnp.float32),
            pltpu.VMEM((4, L), jnp.float32),
            pltpu.VMEM((BLK, D), jnp.float32),
            pltpu.SemaphoreType.DMA,
            pltpu.SemaphoreType.DMA,
        ],
    )
    def k(table_hbm, ids_hbm, warr_hbm, out_hbm, ids_v, rows_a, rows_b, w_v,
          out_v, sem_a, sem_b):
        wid = lax.axis_index("s") * NC + lax.axis_index("c")
        base = wid * rpw

        pltpu.sync_copy(warr_hbm, w_v)
        w0 = w_v[0, :]
        w1 = w_v[1, :]
        w2 = w_v[2, :]
        w3 = w_v[3, :]

        zero = jnp.zeros((L,), jnp.float32)

        def start_gather(rr, buf, sem):
            pltpu.async_copy(
                table_hbm.at[ids_v.at[rr, 0]], buf.at[pl.ds(0, HALF)], sem)
            pltpu.async_copy(
                table_hbm.at[ids_v.at[rr, 1]], buf.at[pl.ds(HALF, HALF)], sem)

        def wait_gather(buf, sem):
            pltpu.make_async_copy(
                table_hbm.at[pl.ds(0, IDS_PAD)], buf, sem).wait()

        def field_sum(buf, lo, hi, unroll):
            n = (hi - lo) // unroll

            def jbody(t, accs):
                j = lo + t * unroll
                accs = list(accs)
                for u in range(unroll):
                    for d in range(ND):
                        accs[d] = accs[d] + buf[j + u, pl.ds(d * L, L)]
                return tuple(accs)

            return lax.fori_loop(0, n, jbody, (zero,) * ND)

        def accum_row(buf, rr):
            s0 = field_sum(buf, 0, 20, 10)
            s1 = field_sum(buf, 20, 40, 10)
            s2 = field_sum(buf, 40, 90, 10)
            s3 = field_sum(buf, 90, NIDS, 10)
            for d in range(ND):
                out_v[rr, pl.ds(d * L, L)] = (
                    s0[d] * w0 + s1[d] * w1 + s2[d] * w2 + s3[d] * w3)

        def pair_body(kk, carry):
            start_gather(2 * kk + 1, rows_b, sem_b)
            wait_gather(rows_a, sem_a)
            accum_row(rows_a, 2 * kk)

            @pl.when(kk < BLK // 2 - 1)
            def _():
                start_gather(2 * kk + 2, rows_a, sem_a)

            wait_gather(rows_b, sem_b)
            accum_row(rows_b, 2 * kk + 1)
            return carry

        def blk_body(blk, carry):
            pltpu.sync_copy(ids_hbm.at[pl.ds(base + blk * BLK, BLK)], ids_v)
            start_gather(0, rows_a, sem_a)
            lax.fori_loop(0, BLK // 2, pair_body, 0)
            pltpu.sync_copy(out_v, out_hbm.at[pl.ds(base + blk * BLK, BLK)])
            return carry

        lax.fori_loop(0, nblk, blk_body, 0)

    return k(table, ids3, warr)


def _tc_normalize(e):
    B = e.shape[0]
    blk = 2048

    def nrm(e_ref, o_ref):
        x = e_ref[...]
        n = jnp.sqrt(jnp.sum(x * x, axis=1, keepdims=True))
        o_ref[...] = x / jnp.maximum(n, 1e-12)

    return pl.pallas_call(
        nrm,
        out_shape=jax.ShapeDtypeStruct((B, D), jnp.float32),
        grid=(B // blk,),
        in_specs=[pl.BlockSpec((blk, D), lambda i: (i, 0))],
        out_specs=pl.BlockSpec((blk, D), lambda i: (i, 0)),
    )(e)


def kernel(tags_ids, component_ids, summary_ids, body_ids, token_embedding,
           field_weights):
    B = tags_ids.shape[0]
    lens = (tags_ids.shape[1], component_ids.shape[1], summary_ids.shape[1],
            body_ids.shape[1])
    pad = IDS_PAD - sum(lens)
    V = token_embedding.shape[0]
    pad_ids = (jnp.arange(B * pad, dtype=jnp.int32) % V).reshape(B, pad)
    ids = jnp.concatenate(
        [tags_ids.astype(jnp.int32), component_ids.astype(jnp.int32),
         summary_ids.astype(jnp.int32), body_ids.astype(jnp.int32),
         pad_ids], axis=1)
    ids3 = ids.reshape(B, 2, HALF)
    scales = field_weights.astype(jnp.float32) / jnp.array(lens, jnp.float32)
    warr = jnp.broadcast_to(scales[:, None], (4, L))
    e = _sc_embed(token_embedding, ids3, warr, B)
    return _tc_normalize(e)
